# baseline (speedup 1.0000x reference)
import functools

import jax
import jax.numpy as jnp
from jax import lax
from jax.experimental import pallas as pl
from jax.experimental.pallas import tpu as pltpu
from jax.experimental.pallas import tpu_sc as plsc

H = 128
B = 16
N = 16384
NB = 2048
VT = 12544


def _last_body(batch_ref, out_ref):
    batch = batch_ref[:, :]
    seg = lax.broadcasted_iota(jnp.int32, (B, N), 0)
    pos = lax.broadcasted_iota(jnp.int32, (B, N), 1)
    masked = jnp.where(batch == seg, pos, -1)
    last = jnp.max(masked, axis=1, keepdims=True)
    out_ref[:, :] = jnp.clip(last, 0, N - 1)


def _vn_body(idx_hbm, x_hbm, out_hbm, idx_v, rows_v, sem):
    cid = lax.axis_index("c")
    sid = lax.axis_index("s")

    @pl.when(jnp.logical_and(cid == 0, sid == 0))
    def _():
        pltpu.sync_copy(idx_hbm, idx_v)
        pltpu.async_copy(x_hbm.at[idx_v], rows_v, sem).wait()
        pltpu.sync_copy(rows_v, out_hbm)


_vn_kernel = functools.partial(
    pl.kernel,
    out_type=jax.ShapeDtypeStruct((B, H), jnp.float32),
    mesh=plsc.VectorSubcoreMesh(core_axis_name="c", subcore_axis_name="s"),
    scratch_types=[
        pltpu.VMEM((B,), jnp.int32),
        pltpu.VMEM((B, H), jnp.float32),
        pltpu.SemaphoreType.DMA,
    ],
)(_vn_body)


def _prep(v_n_ref, batch_ref, nc_ref, x_ref, w1_ref, b1_ref, w2_ref, b2_ref,
          qwt_ref, qb_ref, w3_ref, b3_ref, s_h_ref):
    batch = batch_ref[:, :]
    seg = lax.broadcasted_iota(jnp.int32, (B, N), 0)
    onehot_f = (batch == seg).astype(jnp.float32)
    v_n = v_n_ref[:, :]

    c = (jnp.dot(v_n, w1_ref[:, :], preferred_element_type=jnp.float32)
         + b1_ref[:, :] + b2_ref[:, :])
    qb = qb_ref[0, 0]
    w2 = w2_ref[:, :]
    qwt = qwt_ref[:, :]

    s_g = jnp.zeros((B, H), dtype=jnp.float32)
    for k in range(N // NB):
        sl = pl.ds(k * NB, NB)
        xk = x_ref[sl, :]
        oh_k = onehot_f[:, k * NB:(k + 1) * NB]
        cb_k = lax.dot_general(oh_k, c, (((0,), (0,)), ((), ())),
                               preferred_element_type=jnp.float32)
        pre = jnp.dot(xk, w2, preferred_element_type=jnp.float32) + cb_k
        sg = jax.nn.sigmoid(pre)
        alpha = lax.dot_general(qwt, sg, (((1,), (1,)), ((), ())),
                                preferred_element_type=jnp.float32) + qb
        wk = nc_ref[:, k * NB:(k + 1) * NB] * alpha
        a_k = oh_k * wk
        s_g = s_g + jnp.dot(a_k, xk, preferred_element_type=jnp.float32)

    s_h_ref[:, :] = (
        jnp.dot(v_n, w3_ref[0:H, :], preferred_element_type=jnp.float32)
        + jnp.dot(s_g, w3_ref[H:2 * H, :], preferred_element_type=jnp.float32)
        + b3_ref[:, :])


def _body(v_n_ref, batch_ref, nc_ref, x_ref, w1_ref, b1_ref, w2_ref, b2_ref,
          qwt_ref, qb_ref, w3_ref, b3_ref, tbl_ref, out_ref, s_h_ref):
    @pl.when(pl.program_id(0) == 0)
    def _():
        _prep(v_n_ref, batch_ref, nc_ref, x_ref, w1_ref, b1_ref, w2_ref,
              b2_ref, qwt_ref, qb_ref, w3_ref, b3_ref, s_h_ref)

    out_ref[:, :] = lax.dot_general(
        s_h_ref[:, :], tbl_ref[:, :], (((1,), (1,)), ((), ())),
        preferred_element_type=jnp.float32)


@jax.jit
def kernel(node_embedding, item_embedding_table, batch, num_count,
           W1, b1, W2, b2, qw, qb, W3, b3):
    n, h = node_embedding.shape
    v = item_embedding_table.shape[0]
    batch_row = batch.astype(jnp.int32).reshape(1, n)
    nc_row = num_count.reshape(1, n)

    last_idx = pl.pallas_call(
        _last_body,
        out_shape=jax.ShapeDtypeStruct((B, 1), jnp.int32),
    )(batch_row)
    v_n = _vn_kernel(last_idx.reshape(B), node_embedding)

    const = lambda i: (0, 0)
    grid = (v + VT - 1) // VT
    z = pl.pallas_call(
        _body,
        grid=(grid,),
        in_specs=[
            pl.BlockSpec((B, h), const),
            pl.BlockSpec((1, n), const),
            pl.BlockSpec((1, n), const),
            pl.BlockSpec((n, h), const),
            pl.BlockSpec((h, h), const),
            pl.BlockSpec((1, h), const),
            pl.BlockSpec((h, h), const),
            pl.BlockSpec((1, h), const),
            pl.BlockSpec((1, h), const),
            pl.BlockSpec((1, 1), const),
            pl.BlockSpec((2 * h, h), const),
            pl.BlockSpec((1, h), const),
            pl.BlockSpec((VT, h), lambda i: (i, 0)),
        ],
        out_specs=pl.BlockSpec((B, VT), lambda i: (0, i)),
        out_shape=jax.ShapeDtypeStruct((B, v), jnp.float32),
        scratch_shapes=[pltpu.VMEM((B, h), jnp.float32)],
    )(v_n, batch_row, nc_row, node_embedding,
      W1, b1.reshape(1, h), W2, b2.reshape(1, h),
      qw.reshape(1, h), qb.reshape(1, 1), W3, b3.reshape(1, h),
      item_embedding_table)
    return z
